# baseline (speedup 1.0000x reference)
import functools

import jax
import jax.numpy as jnp
import numpy as np
from jax import lax
from jax.experimental import pallas as pl
from jax.experimental.pallas import tpu as pltpu
from jax.experimental.pallas import tpu_sc as plsc

N = 10000
F_IN = 128
H = 16
PK = 128 // H
NP = N // PK
BN_EPS = 1e-5
BN_SCALE = float(1.0 / np.sqrt(1.0 + BN_EPS))

NC = 2
NS = 16
NW = NC * NS
EPW = 10000
ECH = 2000
NCH = EPW // ECH
N_ACC = 10240
ZROWS = N_ACC // NS
OROWS = 624
OTAIL = N - NS * OROWS
NBUF = 3


def _segment_sum_sc(y, src, dst):
    mesh = plsc.VectorSubcoreMesh(core_axis_name="c", subcore_axis_name="s")

    @functools.partial(
        pl.kernel,
        out_type=(jax.ShapeDtypeStruct((N, H), jnp.float32),
                  jax.ShapeDtypeStruct((N, H), jnp.float32)),
        mesh=mesh,
        scratch_types=[
            pltpu.VMEM((EPW,), jnp.int32),
            pltpu.VMEM((EPW,), jnp.int32),
            [pltpu.VMEM((ECH, H), jnp.float32)] * NBUF,
            pltpu.VMEM((128, H), jnp.float32),
            pltpu.VMEM_SHARED((N_ACC, H), jnp.float32),
            [pltpu.SemaphoreType.DMA] * NBUF,
            [pltpu.SemaphoreType.DMA] * NBUF,
        ],
        compiler_params=pltpu.CompilerParams(use_tc_tiling_on_sc=False),
    )
    def kern(y_hbm, src_hbm, dst_hbm, out0, out1,
             src_v, dst_v, bufs, zbuf, acc, gsems, ssems):
        c = lax.axis_index("c")
        s = lax.axis_index("s")
        w = c * NS + s

        pltpu.sync_copy(src_hbm.at[pl.ds(w * EPW, EPW)], src_v)
        pltpu.sync_copy(dst_hbm.at[pl.ds(w * EPW, EPW)], dst_v)

        zero_row = jnp.zeros((H,), jnp.float32)

        def zero_body(i, carry):
            zbuf[i, :] = zero_row
            return carry

        lax.fori_loop(0, 128, zero_body, 0)
        for t in range(ZROWS // 128):
            pltpu.sync_copy(zbuf, acc.at[pl.ds(s * ZROWS + t * 128, 128)])
        plsc.subcore_barrier()

        def sg(j, b):
            pltpu.async_copy(
                y_hbm.at[src_v.at[pl.ds(j * ECH, ECH)]], bufs[b], gsems[b])

        def dg(j, b):
            pltpu.make_async_copy(
                y_hbm.at[src_v.at[pl.ds(j * ECH, ECH)]], bufs[b],
                gsems[b]).wait()

        def ss(j, b):
            pltpu.async_copy(
                bufs[b], acc.at[dst_v.at[pl.ds(j * ECH, ECH)]], ssems[b],
                add=True)

        def ws(j, b):
            pltpu.make_async_copy(
                bufs[b], acc.at[dst_v.at[pl.ds(j * ECH, ECH)]],
                ssems[b]).wait()

        sg(0, 0)
        sg(1, 1)
        dg(0, 0); ss(0, 0); sg(2, 2)
        dg(1, 1); ss(1, 1); ws(0, 0); sg(3, 0)
        dg(2, 2); ss(2, 2); ws(1, 1); sg(4, 1)
        dg(3, 0); ss(3, 0); ws(2, 2)
        dg(4, 1); ss(4, 1); ws(3, 0)
        ws(4, 1)
        plsc.subcore_barrier()

        @pl.when(c == 0)
        def _copy_out0():
            pltpu.sync_copy(acc.at[pl.ds(s * OROWS, OROWS)],
                            out0.at[pl.ds(s * OROWS, OROWS)])

            @pl.when(s == NS - 1)
            def _tail0():
                pltpu.sync_copy(acc.at[pl.ds(NS * OROWS, OTAIL)],
                                out0.at[pl.ds(NS * OROWS, OTAIL)])

        @pl.when(c == 1)
        def _copy_out1():
            pltpu.sync_copy(acc.at[pl.ds(s * OROWS, OROWS)],
                            out1.at[pl.ds(s * OROWS, OROWS)])

            @pl.when(s == NS - 1)
            def _tail1():
                pltpu.sync_copy(acc.at[pl.ds(NS * OROWS, OTAIL)],
                                out1.at[pl.ds(NS * OROWS, OTAIL)])

    return kern(y, src, dst)


_BR = NP
_GRID = 1


def _elu(t):
    return jnp.where(t > 0.0, t, jnp.exp(jnp.minimum(t, 0.0)) - 1.0)


def _tc1(x2, KW1a):
    def body(x_ref, w_ref, o_ref):
        o_ref[...] = jnp.dot(x_ref[...], w_ref[...],
                             preferred_element_type=jnp.float32)

    return pl.pallas_call(
        body,
        grid=(_GRID,),
        in_specs=[
            pl.BlockSpec((_BR, PK * F_IN), lambda i: (i, 0)),
            pl.BlockSpec((PK * F_IN, 128), lambda i: (0, 0)),
        ],
        out_specs=pl.BlockSpec((_BR, 128), lambda i: (i, 0)),
        out_shape=jax.ShapeDtypeStruct((NP, 128), jnp.float32),
    )(x2, KW1a)


def _tc2(y1, p0, p1, b1a, g1, be1, KW1b, b1b, KW2a):
    def body(y_ref, p0_ref, p1_ref, ba_ref, g_ref, be_ref, kwb_ref, bb_ref,
             kwa2_ref, h1_ref, y2_ref):
        t = y_ref[...] + p0_ref[...] + p1_ref[...] + ba_ref[...]
        t = g_ref[...] * (t * BN_SCALE) + be_ref[...]
        t = _elu(t)
        h1 = _elu(jnp.dot(t, kwb_ref[...], preferred_element_type=jnp.float32)
                  + bb_ref[...])
        h1_ref[...] = h1
        y2_ref[...] = jnp.dot(h1, kwa2_ref[...],
                              preferred_element_type=jnp.float32)

    vec = pl.BlockSpec((1, 128), lambda i: (0, 0))
    mat = pl.BlockSpec((128, 128), lambda i: (0, 0))
    row = pl.BlockSpec((_BR, 128), lambda i: (i, 0))
    return pl.pallas_call(
        body,
        grid=(_GRID,),
        in_specs=[row, row, row, vec, vec, vec, mat, vec, mat],
        out_specs=[row, row],
        out_shape=[jax.ShapeDtypeStruct((NP, 128), jnp.float32),
                   jax.ShapeDtypeStruct((NP, 128), jnp.float32)],
    )(y1, p0, p1, b1a, g1, be1, KW1b, b1b, KW2a)


def _tc3(y2, p0, p1, h1, b2a, g2, be2, KW2b, b2b, KWl1h1, KWl1h2, bl1, KWl2,
         bl2):
    def body(y_ref, p0_ref, p1_ref, h1_ref, ba_ref, g_ref, be_ref, kwb_ref,
             bb_ref, kwl1a_ref, kwl1b_ref, bl1_ref, kwl2_ref, bl2_ref, o_ref):
        t = y_ref[...] + p0_ref[...] + p1_ref[...] + ba_ref[...]
        t = g_ref[...] * (t * BN_SCALE) + be_ref[...]
        t = _elu(t)
        h2 = _elu(jnp.dot(t, kwb_ref[...], preferred_element_type=jnp.float32)
                  + bb_ref[...])
        z = (jnp.dot(h1_ref[...], kwl1a_ref[...],
                     preferred_element_type=jnp.float32)
             + jnp.dot(h2, kwl1b_ref[...], preferred_element_type=jnp.float32)
             + bl1_ref[...])
        z = jnp.maximum(z, 0.0)
        o_ref[...] = (jnp.dot(z, kwl2_ref[...],
                              preferred_element_type=jnp.float32)
                      + bl2_ref[...])

    vec = pl.BlockSpec((1, 128), lambda i: (0, 0))
    mat = pl.BlockSpec((128, 128), lambda i: (0, 0))
    row = pl.BlockSpec((_BR, 128), lambda i: (i, 0))
    return pl.pallas_call(
        body,
        grid=(_GRID,),
        in_specs=[row, row, row, row, vec, vec, vec, mat, vec, mat, mat, vec,
                  mat, vec],
        out_specs=row,
        out_shape=jax.ShapeDtypeStruct((NP, 128), jnp.float32),
    )(y2, p0, p1, h1, b2a, g2, be2, KW2b, b2b, KWl1h1, KWl1h2, bl1, KWl2, bl2)


def kernel(x, edge_index, W1a, b1a, g1, be1, W1b, b1b,
           W2a, b2a, g2, be2, W2b, b2b, Wl1, bl1, Wl2, bl2):
    eye8 = jnp.eye(PK, dtype=jnp.float32)
    kr = lambda W: jnp.kron(eye8, W)
    tl = lambda v: jnp.tile(v, PK).reshape(1, 128)

    src = edge_index[0]
    dst = edge_index[1]
    y1 = _tc1(x.reshape(NP, PK * F_IN), kr(W1a))
    p1a, p1b = _segment_sum_sc(y1.reshape(N, H), src, dst)
    h1, y2 = _tc2(y1, p1a.reshape(NP, 128), p1b.reshape(NP, 128),
                  tl(b1a), tl(g1), tl(be1), kr(W1b), tl(b1b), kr(W2a))
    p2a, p2b = _segment_sum_sc(y2.reshape(N, H), src, dst)
    outp = _tc3(y2, p2a.reshape(NP, 128), p2b.reshape(NP, 128), h1,
                tl(b2a), tl(g2), tl(be2), kr(W2b), tl(b2b),
                kr(Wl1[:H]), kr(Wl1[H:]), tl(bl1), kr(Wl2), tl(bl2))
    return outp.reshape(N, H)


# speedup vs baseline: 1.1386x; 1.1386x over previous
import functools

import jax
import jax.numpy as jnp
import numpy as np
from jax import lax
from jax.experimental import pallas as pl
from jax.experimental.pallas import tpu as pltpu
from jax.experimental.pallas import tpu_sc as plsc

N = 10000
F_IN = 128
H = 16
PK = 128 // H
NP = N // PK
BN_EPS = 1e-5
BN_SCALE = float(1.0 / np.sqrt(1.0 + BN_EPS))

NC = 2
NS = 16
NW = NC * NS
EPW = 10000
ECH = 2000
NCH = EPW // ECH
N_ACC = 10240
ZROWS = N_ACC // NS
OROWS = 624
OTAIL = N - NS * OROWS
NBUF = 3


def _segment_sum_sc(y, edge_index):
    mesh = plsc.VectorSubcoreMesh(core_axis_name="c", subcore_axis_name="s")

    @functools.partial(
        pl.kernel,
        out_type=(jax.ShapeDtypeStruct((N, H), jnp.float32),
                  jax.ShapeDtypeStruct((N, H), jnp.float32)),
        mesh=mesh,
        scratch_types=[
            pltpu.VMEM((EPW,), jnp.int32),
            pltpu.VMEM((EPW,), jnp.int32),
            [pltpu.VMEM((ECH, H), jnp.float32)] * NBUF,
            pltpu.VMEM((128, H), jnp.float32),
            pltpu.VMEM_SHARED((N_ACC, H), jnp.float32),
            [pltpu.SemaphoreType.DMA] * NBUF,
            [pltpu.SemaphoreType.DMA] * NBUF,
        ],
        compiler_params=pltpu.CompilerParams(use_tc_tiling_on_sc=False),
    )
    def kern(y_hbm, ei_hbm, out0, out1,
             src_v, dst_v, bufs, zbuf, acc, gsems, ssems):
        c = lax.axis_index("c")
        s = lax.axis_index("s")
        w = c * NS + s

        pltpu.sync_copy(ei_hbm.at[0, pl.ds(w * EPW, EPW)], src_v)
        pltpu.sync_copy(ei_hbm.at[1, pl.ds(w * EPW, EPW)], dst_v)

        zero_row = jnp.zeros((H,), jnp.float32)

        def zero_body(i, carry):
            zbuf[i, :] = zero_row
            return carry

        lax.fori_loop(0, 128, zero_body, 0)
        for t in range(ZROWS // 128):
            pltpu.sync_copy(zbuf, acc.at[pl.ds(s * ZROWS + t * 128, 128)])
        plsc.subcore_barrier()

        def sg(j, b):
            pltpu.async_copy(
                y_hbm.at[src_v.at[pl.ds(j * ECH, ECH)]], bufs[b], gsems[b])

        def dg(j, b):
            pltpu.make_async_copy(
                y_hbm.at[src_v.at[pl.ds(j * ECH, ECH)]], bufs[b],
                gsems[b]).wait()

        def ss(j, b):
            pltpu.async_copy(
                bufs[b], acc.at[dst_v.at[pl.ds(j * ECH, ECH)]], ssems[b],
                add=True)

        def ws(j, b):
            pltpu.make_async_copy(
                bufs[b], acc.at[dst_v.at[pl.ds(j * ECH, ECH)]],
                ssems[b]).wait()

        sg(0, 0)
        sg(1, 1)
        dg(0, 0); ss(0, 0); sg(2, 2)
        dg(1, 1); ss(1, 1); ws(0, 0); sg(3, 0)
        dg(2, 2); ss(2, 2); ws(1, 1); sg(4, 1)
        dg(3, 0); ss(3, 0); ws(2, 2)
        dg(4, 1); ss(4, 1); ws(3, 0)
        ws(4, 1)
        plsc.subcore_barrier()

        @pl.when(c == 0)
        def _copy_out0():
            pltpu.sync_copy(acc.at[pl.ds(s * OROWS, OROWS)],
                            out0.at[pl.ds(s * OROWS, OROWS)])

            @pl.when(s == NS - 1)
            def _tail0():
                pltpu.sync_copy(acc.at[pl.ds(NS * OROWS, OTAIL)],
                                out0.at[pl.ds(NS * OROWS, OTAIL)])

        @pl.when(c == 1)
        def _copy_out1():
            pltpu.sync_copy(acc.at[pl.ds(s * OROWS, OROWS)],
                            out1.at[pl.ds(s * OROWS, OROWS)])

            @pl.when(s == NS - 1)
            def _tail1():
                pltpu.sync_copy(acc.at[pl.ds(NS * OROWS, OTAIL)],
                                out1.at[pl.ds(NS * OROWS, OTAIL)])

    return kern(y, edge_index)


_BR = NP
_GRID = 1


def _elu(t):
    return jnp.where(t > 0.0, t, jnp.exp(jnp.minimum(t, 0.0)) - 1.0)


def _tc1(x, W1a_ext):
    def body(x_ref, w_ref, o_ref):
        z = jnp.dot(x_ref[...], w_ref[...],
                    preferred_element_type=jnp.float32)
        z3 = z.reshape(NP, PK, 128)
        lg = lax.broadcasted_iota(jnp.int32, (NP, 128), 1) // H
        o = jnp.where(lg == 0, z3[:, 0, :], z3[:, 1, :])
        for g in range(2, PK):
            o = jnp.where(lg == g, z3[:, g, :], o)
        o_ref[...] = o

    return pl.pallas_call(
        body,
        grid=(_GRID,),
        in_specs=[
            pl.BlockSpec((PK * _BR, F_IN), lambda i: (i, 0)),
            pl.BlockSpec((F_IN, 128), lambda i: (0, 0)),
        ],
        out_specs=pl.BlockSpec((_BR, 128), lambda i: (i, 0)),
        out_shape=jax.ShapeDtypeStruct((NP, 128), jnp.float32),
    )(x, W1a_ext)


def _tc2(y1, p0, p1, b1a, g1, be1, KW1b, b1b, KW2a):
    def body(y_ref, p0_ref, p1_ref, ba_ref, g_ref, be_ref, kwb_ref, bb_ref,
             kwa2_ref, h1_ref, y2_ref):
        t = y_ref[...] + p0_ref[...] + p1_ref[...] + ba_ref[...]
        t = g_ref[...] * (t * BN_SCALE) + be_ref[...]
        t = _elu(t)
        h1 = _elu(jnp.dot(t, kwb_ref[...], preferred_element_type=jnp.float32)
                  + bb_ref[...])
        h1_ref[...] = h1
        y2_ref[...] = jnp.dot(h1, kwa2_ref[...],
                              preferred_element_type=jnp.float32)

    vec = pl.BlockSpec((1, 128), lambda i: (0, 0))
    mat = pl.BlockSpec((128, 128), lambda i: (0, 0))
    row = pl.BlockSpec((_BR, 128), lambda i: (i, 0))
    return pl.pallas_call(
        body,
        grid=(_GRID,),
        in_specs=[row, row, row, vec, vec, vec, mat, vec, mat],
        out_specs=[row, row],
        out_shape=[jax.ShapeDtypeStruct((NP, 128), jnp.float32),
                   jax.ShapeDtypeStruct((NP, 128), jnp.float32)],
    )(y1, p0, p1, b1a, g1, be1, KW1b, b1b, KW2a)


def _tc3(y2, p0, p1, h1, b2a, g2, be2, KW2b, b2b, KWl1h1, KWl1h2, bl1, KWl2,
         bl2):
    def body(y_ref, p0_ref, p1_ref, h1_ref, ba_ref, g_ref, be_ref, kwb_ref,
             bb_ref, kwl1a_ref, kwl1b_ref, bl1_ref, kwl2_ref, bl2_ref, o_ref):
        t = y_ref[...] + p0_ref[...] + p1_ref[...] + ba_ref[...]
        t = g_ref[...] * (t * BN_SCALE) + be_ref[...]
        t = _elu(t)
        h2 = _elu(jnp.dot(t, kwb_ref[...], preferred_element_type=jnp.float32)
                  + bb_ref[...])
        z = (jnp.dot(h1_ref[...], kwl1a_ref[...],
                     preferred_element_type=jnp.float32)
             + jnp.dot(h2, kwl1b_ref[...], preferred_element_type=jnp.float32)
             + bl1_ref[...])
        z = jnp.maximum(z, 0.0)
        o_ref[...] = (jnp.dot(z, kwl2_ref[...],
                              preferred_element_type=jnp.float32)
                      + bl2_ref[...])

    vec = pl.BlockSpec((1, 128), lambda i: (0, 0))
    mat = pl.BlockSpec((128, 128), lambda i: (0, 0))
    row = pl.BlockSpec((_BR, 128), lambda i: (i, 0))
    return pl.pallas_call(
        body,
        grid=(_GRID,),
        in_specs=[row, row, row, row, vec, vec, vec, mat, vec, mat, mat, vec,
                  mat, vec],
        out_specs=row,
        out_shape=jax.ShapeDtypeStruct((NP, 128), jnp.float32),
    )(y2, p0, p1, h1, b2a, g2, be2, KW2b, b2b, KWl1h1, KWl1h2, bl1, KWl2, bl2)


def kernel(x, edge_index, W1a, b1a, g1, be1, W1b, b1b,
           W2a, b2a, g2, be2, W2b, b2b, Wl1, bl1, Wl2, bl2):
    eye8 = jnp.eye(PK, dtype=jnp.float32)
    kr = lambda W: jnp.kron(eye8, W)
    tl = lambda v: jnp.tile(v, PK).reshape(1, 128)

    y1 = _tc1(x, jnp.tile(W1a, (1, PK)))
    p1a, p1b = _segment_sum_sc(y1.reshape(N, H), edge_index)
    h1, y2 = _tc2(y1, p1a.reshape(NP, 128), p1b.reshape(NP, 128),
                  tl(b1a), tl(g1), tl(be1), kr(W1b), tl(b1b), kr(W2a))
    p2a, p2b = _segment_sum_sc(y2.reshape(N, H), edge_index)
    outp = _tc3(y2, p2a.reshape(NP, 128), p2b.reshape(NP, 128), h1,
                tl(b2a), tl(g2), tl(be2), kr(W2b), tl(b2b),
                kr(Wl1[:H]), kr(Wl1[H:]), tl(bl1), kr(Wl2), tl(bl2))
    return outp.reshape(N, H)
